# baseline (speedup 1.0000x reference)
import dataclasses
import functools

import jax
import jax.numpy as jnp
from jax import lax
from jax.experimental import pallas as pl
from jax.experimental.pallas import tpu as pltpu
from jax.experimental.pallas import tpu_sc as plsc

F32 = jnp.float32
N = 50000
E = 800000
NO = 32
LEAK = 0.1
EPS = 1e-5

NPAD = 50048
NW = 32
GW = 125
NWIN = E // GW
WPW = NWIN // NW
EPW = E // NW
CH = 1000
NCH = EPW // CH
BE = 4000
BN = 2000
CNT_CB = 2944


def _mesh():
    return plsc.VectorSubcoreMesh(core_axis_name="core", subcore_axis_name="subcore",
                                  num_cores=2, num_subcores=16)


def _sc_params():
    cp = pltpu.CompilerParams()
    fields = pltpu.CompilerParams.__dataclass_fields__
    if "needs_layout_passes" in fields:
        cp = dataclasses.replace(cp, needs_layout_passes=False)
    if "use_tc_tiling_on_sc" in fields:
        cp = dataclasses.replace(cp, use_tc_tiling_on_sc=False)
    return cp


def _leaky(x, s):
    return jnp.where(x >= 0, x, s * x)


def _sc_counts(src, dst):
    out_t = (jax.ShapeDtypeStruct((NW, NPAD), F32),
             jax.ShapeDtypeStruct((NW, NPAD), F32))

    @functools.partial(
        pl.kernel, out_type=out_t, mesh=_mesh(), compiler_params=_sc_params(),
        scratch_types=[pltpu.VMEM((NPAD,), F32), pltpu.VMEM((NPAD,), F32),
                       pltpu.VMEM((CH,), jnp.int32), pltpu.VMEM((CH,), jnp.int32)])
    def k(src_h, dst_h, cs_h, cd_h, cs_v, cd_v, s_v, d_v):
        wid = lax.axis_index("subcore") * 2 + lax.axis_index("core")
        zero = jnp.zeros((16,), F32)

        @pl.loop(0, NPAD, step=16)
        def _(i):
            cs_v[pl.ds(i, 16)] = zero
            cd_v[pl.ds(i, 16)] = zero

        base = wid * EPW
        ones = jnp.ones((16,), F32)
        tail = jnp.where(lax.iota(jnp.int32, 16) >= 8, 1.0, 0.0).astype(F32)

        @pl.loop(0, NCH)
        def _(j):
            pltpu.sync_copy(src_h.at[pl.ds(base + j * CH, CH)], s_v)
            pltpu.sync_copy(dst_h.at[pl.ds(base + j * CH, CH)], d_v)

            @pl.loop(0, CH - 16, step=16)
            def _(i):
                plsc.addupdate_scatter(cs_v, [s_v[pl.ds(i, 16)]], ones)
                plsc.addupdate_scatter(cd_v, [d_v[pl.ds(i, 16)]], ones)

            plsc.addupdate_scatter(cs_v, [s_v[pl.ds(CH - 16, 16)]], tail)
            plsc.addupdate_scatter(cd_v, [d_v[pl.ds(CH - 16, 16)]], tail)

        pltpu.sync_copy(cs_v, cs_h.at[wid])
        pltpu.sync_copy(cd_v, cd_h.at[wid])

    return k(src, dst)


def _sc_gather(xp, src2, dst2):
    D = xp.shape[1]
    out_t = (jax.ShapeDtypeStruct((E, D), F32), jax.ShapeDtypeStruct((E, D), F32))

    WPS = 2

    @functools.partial(pl.kernel, out_type=out_t, mesh=_mesh(),
                       compiler_params=_sc_params(),
                       scratch_types=[pltpu.SemaphoreType.DMA,
                                      pltpu.SemaphoreType.DMA])
    def k(x_h, s_h, d_h, xs_h, xd_h, sem_s, sem_d):
        def body(s_v, d_v, xs_v, xd_v):
            cps = []
            for w in range(WPS):
                cps.append(pltpu.async_copy(
                    x_h.at[s_v.at[w]], xs_v.at[pl.ds(w * GW, GW)], sem_s))
                cps.append(pltpu.async_copy(
                    x_h.at[d_v.at[w]], xd_v.at[pl.ds(w * GW, GW)], sem_d))
            for c in cps:
                c.wait()

        pltpu.emit_pipeline(
            body, grid=(NWIN // WPS,),
            in_specs=[pl.BlockSpec((WPS, GW), lambda i: (i, 0)),
                      pl.BlockSpec((WPS, GW), lambda i: (i, 0))],
            out_specs=[pl.BlockSpec((WPS * GW, D), lambda i: (i, 0)),
                       pl.BlockSpec((WPS * GW, D), lambda i: (i, 0))],
            core_axis_name=("core", "subcore"),
            dimension_semantics=(pltpu.PARALLEL,),
        )(s_h, d_h, xs_h, xd_h)

    return k(xp, src2, dst2)


def _sc_scatter_rows(m, dst2, zrows):
    out_t = jax.ShapeDtypeStruct((2, NPAD, NO), F32)
    SCH = 500
    SNCH = EPW // SCH

    @functools.partial(
        pl.kernel, out_type=out_t, mesh=_mesh(), compiler_params=_sc_params(),
        scratch_types=[pltpu.VMEM_SHARED((NPAD, NO), F32),
                       pltpu.VMEM((SCH, NO), F32),
                       pltpu.VMEM((4, GW), jnp.int32)])
    def k(m_h, d2_h, z_h, part_h, acc, m_v, i2_v):
        cid = lax.axis_index("core")
        sid = lax.axis_index("subcore")
        wid = sid * 2 + cid
        rps = NPAD // 16

        pltpu.sync_copy(z_h, acc.at[pl.ds(sid * rps, rps)])
        plsc.subcore_barrier()

        base = wid * EPW
        rbase = wid * WPW

        @pl.loop(0, SNCH)
        def _(j):
            pltpu.sync_copy(m_h.at[pl.ds(base + j * SCH, SCH)], m_v)
            pltpu.sync_copy(d2_h.at[pl.ds(rbase + j * 4, 4)], i2_v)
            for r in range(4):
                pltpu.sync_copy(m_v.at[pl.ds(r * GW, GW)], acc.at[i2_v.at[r]],
                                add=True)

        plsc.subcore_barrier()
        pltpu.sync_copy(acc.at[pl.ds(sid * rps, rps)],
                        part_h.at[cid, pl.ds(sid * rps, rps)])

    return k(m, dst2, zrows)


def _sc_scatter_den(ex, dst1):
    out_t = jax.ShapeDtypeStruct((NW, NPAD), F32)

    @functools.partial(
        pl.kernel, out_type=out_t, mesh=_mesh(), compiler_params=_sc_params(),
        scratch_types=[pltpu.VMEM((NPAD,), F32),
                       pltpu.VMEM((CH,), jnp.int32),
                       pltpu.VMEM((CH,), F32)])
    def k(ex_h, d1_h, denp_h, den_v, d_v, e_v):
        wid = lax.axis_index("subcore") * 2 + lax.axis_index("core")
        zero = jnp.zeros((16,), F32)

        @pl.loop(0, NPAD, step=16)
        def _(i):
            den_v[pl.ds(i, 16)] = zero

        base = wid * EPW
        iota = lax.iota(jnp.int32, 16)

        @pl.loop(0, NCH)
        def _(j):
            eoff = base + j * CH
            pltpu.sync_copy(d1_h.at[pl.ds(eoff, CH)], d_v)
            pltpu.sync_copy(ex_h.at[pl.ds(eoff, CH)], e_v)

            @pl.loop(0, CH - 16, step=16)
            def _(i):
                plsc.addupdate_scatter(den_v, [d_v[pl.ds(i, 16)]],
                                       e_v[pl.ds(i, 16)])

            tv = jnp.where(iota >= 8, e_v[pl.ds(CH - 16, 16)], 0.0)
            plsc.addupdate_scatter(den_v, [d_v[pl.ds(CH - 16, 16)]], tv)

        pltpu.sync_copy(den_v, denp_h.at[wid])

    return k(ex, dst1)


def _colstats(x, rows=None):
    rows = BE if rows is None else rows
    D = x.shape[1]
    T = x.shape[0] // rows

    def body(x_ref, o_ref):
        @pl.when(pl.program_id(0) == 0)
        def _():
            o_ref[...] = jnp.zeros_like(o_ref)

        xv = x_ref[...]
        o_ref[...] += jnp.concatenate(
            [jnp.sum(xv, 0, keepdims=True), jnp.sum(xv * xv, 0, keepdims=True)], 0)

    return pl.pallas_call(
        body, grid=(T,),
        in_specs=[pl.BlockSpec((rows, D), lambda i: (i, 0))],
        out_specs=pl.BlockSpec((2, D), lambda i: (0, 0)),
        out_shape=jax.ShapeDtypeStruct((2, D), F32))(x)


def _cnt_sum(cs_p, cd_p):
    CB = CNT_CB
    T = NPAD // CB

    def body(cs_ref, cd_ref, o_ref):
        o_ref[...] = jnp.concatenate(
            [jnp.sum(cs_ref[...], 0, keepdims=True),
             jnp.sum(cd_ref[...], 0, keepdims=True)], 0)

    return pl.pallas_call(
        body, grid=(T,),
        in_specs=[pl.BlockSpec((NW, CB), lambda i: (0, i)),
                  pl.BlockSpec((NW, CB), lambda i: (0, i))],
        out_specs=pl.BlockSpec((2, CB), lambda i: (0, i)),
        out_shape=jax.ShapeDtypeStruct((2, NPAD), F32))(cs_p, cd_p)


def _bn_project(x, nstats, g, b, Wg, asr, adr, cnt2, want_gat):
    D = x.shape[1]
    T = N // BN

    def body(x_ref, ns_ref, g_ref, b_ref, w_ref, as_ref, ad_ref, c_ref, *outs):
        xv = x_ref[...]
        mu = ns_ref[0:1] * (1.0 / N)
        var = ns_ref[1:2] * (1.0 / N) - mu * mu
        xp = g_ref[...] * (xv - mu) * lax.rsqrt(var + EPS) + b_ref[...]
        cs = c_ref[...][:, 0:1]
        cd = c_ref[...][:, 1:2]
        xp2 = xp * xp
        xstats = jnp.concatenate([
            jnp.sum(xp * cs, 0, keepdims=True),
            jnp.sum(xp2 * cs, 0, keepdims=True),
            jnp.sum(xp * cd, 0, keepdims=True),
            jnp.sum(xp2 * cd, 0, keepdims=True)], 0)
        if want_gat:
            xp_ref, hx_ref, es_ref, st_ref = outs
            hx = jnp.dot(xp, w_ref[...], preferred_element_type=F32)
            a = (jnp.dot(hx, as_ref[...], preferred_element_type=F32)
                 + jnp.dot(hx, ad_ref[...], preferred_element_type=F32))
            hx_ref[...] = hx
            es_ref[...] = jnp.exp(_leaky(a, 0.2))
        else:
            xp_ref, st_ref = outs
        xp_ref[...] = xp

        @pl.when(pl.program_id(0) == 0)
        def _():
            st_ref[...] = jnp.zeros_like(st_ref)

        st_ref[...] += xstats

    in_specs = [pl.BlockSpec((BN, D), lambda i: (i, 0)),
                pl.BlockSpec((2, D), lambda i: (0, 0)),
                pl.BlockSpec((1, D), lambda i: (0, 0)),
                pl.BlockSpec((1, D), lambda i: (0, 0)),
                pl.BlockSpec((D, NO), lambda i: (0, 0)),
                pl.BlockSpec((NO, 1), lambda i: (0, 0)),
                pl.BlockSpec((NO, 1), lambda i: (0, 0)),
                pl.BlockSpec((BN, 2), lambda i: (i, 0))]
    if want_gat:
        out_specs = [pl.BlockSpec((BN, D), lambda i: (i, 0)),
                     pl.BlockSpec((BN, NO), lambda i: (i, 0)),
                     pl.BlockSpec((BN, 1), lambda i: (i, 0)),
                     pl.BlockSpec((4, D), lambda i: (0, 0))]
        out_shape = (jax.ShapeDtypeStruct((N, D), F32),
                     jax.ShapeDtypeStruct((N, NO), F32),
                     jax.ShapeDtypeStruct((N, 1), F32),
                     jax.ShapeDtypeStruct((4, D), F32))
    else:
        out_specs = [pl.BlockSpec((BN, D), lambda i: (i, 0)),
                     pl.BlockSpec((4, D), lambda i: (0, 0))]
        out_shape = (jax.ShapeDtypeStruct((N, D), F32),
                     jax.ShapeDtypeStruct((4, D), F32))
    return pl.pallas_call(
        body, grid=(T,), in_specs=in_specs, out_specs=out_specs,
        out_shape=out_shape)(x, nstats, g, b, Wg, asr, adr, cnt2)


def _gat_reduce(part, denpT, hx_prev, es_prev, bias):
    T = N // BN

    def body(p_ref, d_ref, hx_ref, es_ref, b_ref, x_ref, st_ref):
        es = es_ref[...]
        num = p_ref[0] + p_ref[1] + es * hx_ref[...]
        den = jnp.sum(d_ref[...], 1, keepdims=True) + es + 1e-16
        xv = _leaky(num / den + b_ref[...], LEAK)
        x_ref[...] = xv

        @pl.when(pl.program_id(0) == 0)
        def _():
            st_ref[...] = jnp.zeros_like(st_ref)

        st_ref[...] += jnp.concatenate(
            [jnp.sum(xv, 0, keepdims=True), jnp.sum(xv * xv, 0, keepdims=True)], 0)

    return pl.pallas_call(
        body, grid=(T,),
        in_specs=[pl.BlockSpec((2, BN, NO), lambda i: (0, i, 0)),
                  pl.BlockSpec((BN, NW), lambda i: (i, 0)),
                  pl.BlockSpec((BN, NO), lambda i: (i, 0)),
                  pl.BlockSpec((BN, 1), lambda i: (i, 0)),
                  pl.BlockSpec((1, NO), lambda i: (0, 0))],
        out_specs=[pl.BlockSpec((BN, NO), lambda i: (i, 0)),
                   pl.BlockSpec((2, NO), lambda i: (0, 0))],
        out_shape=(jax.ShapeDtypeStruct((N, NO), F32),
                   jax.ShapeDtypeStruct((2, NO), F32)))(
        part, denpT, hx_prev, es_prev, bias)


def _edge_mlp1(xs, xd, e, sc3, sh3, W1, b1, Wg, asr, adr, ni, nei, want_gat):
    T = E // BE
    din = 2 * ni + nei

    def body(xs_ref, xd_ref, e_ref, sc_ref, sh_ref, w1_ref, b1_ref, wg_ref,
             as_ref, ad_ref, *outs):
        xsv = xs_ref[...]
        xdv = xd_ref[...]
        ev = e_ref[...]
        sc = sc_ref[...]
        sh = sh_ref[...]
        W1 = w1_ref[...]
        xsn = xsv * sc[:, :ni] + sh[:, :ni]
        xdn = xdv * sc[:, ni:2 * ni] + sh[:, ni:2 * ni]
        en = ev * sc[:, 2 * ni:] + sh[:, 2 * ni:]
        z = (jnp.dot(xsn, W1[:ni], preferred_element_type=F32)
             + jnp.dot(xdn, W1[ni:2 * ni], preferred_element_type=F32)
             + jnp.dot(en, W1[2 * ni:], preferred_element_type=F32)
             + b1_ref[...])
        h1 = _leaky(z, LEAK)
        if want_gat:
            h1_ref, m_ref, ex_ref, st_ref = outs
            hxs = jnp.dot(xsv, wg_ref[...], preferred_element_type=F32)
            hxd = jnp.dot(xdv, wg_ref[...], preferred_element_type=F32)
            a = (jnp.dot(hxs, as_ref[...], preferred_element_type=F32)
                 + jnp.dot(hxd, ad_ref[...], preferred_element_type=F32))
            exv = jnp.exp(_leaky(a, 0.2))
            m_ref[...] = hxs * exv
            ex_ref[...] = exv
        else:
            h1_ref, st_ref = outs
        h1_ref[...] = h1

        @pl.when(pl.program_id(0) == 0)
        def _():
            st_ref[...] = jnp.zeros_like(st_ref)

        st_ref[...] += jnp.concatenate(
            [jnp.sum(h1, 0, keepdims=True), jnp.sum(h1 * h1, 0, keepdims=True)], 0)

    in_specs = [pl.BlockSpec((BE, ni), lambda i: (i, 0)),
                pl.BlockSpec((BE, ni), lambda i: (i, 0)),
                pl.BlockSpec((BE, nei), lambda i: (i, 0)),
                pl.BlockSpec((1, din), lambda i: (0, 0)),
                pl.BlockSpec((1, din), lambda i: (0, 0)),
                pl.BlockSpec((din, NO), lambda i: (0, 0)),
                pl.BlockSpec((1, NO), lambda i: (0, 0)),
                pl.BlockSpec((ni, NO), lambda i: (0, 0)),
                pl.BlockSpec((NO, 1), lambda i: (0, 0)),
                pl.BlockSpec((NO, 1), lambda i: (0, 0))]
    if want_gat:
        out_specs = [pl.BlockSpec((BE, NO), lambda i: (i, 0)),
                     pl.BlockSpec((BE, NO), lambda i: (i, 0)),
                     pl.BlockSpec((BE, 1), lambda i: (i, 0)),
                     pl.BlockSpec((2, NO), lambda i: (0, 0))]
        out_shape = (jax.ShapeDtypeStruct((E, NO), F32),
                     jax.ShapeDtypeStruct((E, NO), F32),
                     jax.ShapeDtypeStruct((E, 1), F32),
                     jax.ShapeDtypeStruct((2, NO), F32))
    else:
        out_specs = [pl.BlockSpec((BE, NO), lambda i: (i, 0)),
                     pl.BlockSpec((2, NO), lambda i: (0, 0))]
        out_shape = (jax.ShapeDtypeStruct((E, NO), F32),
                     jax.ShapeDtypeStruct((2, NO), F32))
    return pl.pallas_call(
        body, grid=(T,), in_specs=in_specs, out_specs=out_specs,
        out_shape=out_shape)(xs, xd, e, sc3, sh3, W1, b1, Wg, asr, adr)


def _edge_mm(h, sc, sh, W, b, do_leaky, do_stats, head=None):
    T = E // BE
    Din = h.shape[1]
    Dout = W.shape[1] if head is None else head[0].shape[1]

    def body(h_ref, sc_ref, sh_ref, w_ref, b_ref, *rest):
        if head is None:
            o_ref = rest[0]
            st = rest[1:]
        else:
            wp_ref, bp_ref, o_ref = rest[0], rest[1], rest[2]
            st = rest[3:]
        hn = h_ref[...] * sc_ref[...] + sh_ref[...]
        z = jnp.dot(hn, w_ref[...], preferred_element_type=F32) + b_ref[...]
        if do_leaky:
            z = _leaky(z, LEAK)
        if head is not None:
            z = jnp.dot(z, wp_ref[...], preferred_element_type=F32) + bp_ref[...]
        o_ref[...] = z
        if do_stats:
            st_ref = st[0]

            @pl.when(pl.program_id(0) == 0)
            def _():
                st_ref[...] = jnp.zeros_like(st_ref)

            st_ref[...] += jnp.concatenate(
                [jnp.sum(z, 0, keepdims=True), jnp.sum(z * z, 0, keepdims=True)], 0)

    in_specs = [pl.BlockSpec((BE, Din), lambda i: (i, 0)),
                pl.BlockSpec((1, Din), lambda i: (0, 0)),
                pl.BlockSpec((1, Din), lambda i: (0, 0)),
                pl.BlockSpec((Din, W.shape[1]), lambda i: (0, 0)),
                pl.BlockSpec((1, W.shape[1]), lambda i: (0, 0))]
    args = [h, sc, sh, W, b]
    if head is not None:
        in_specs += [pl.BlockSpec(head[0].shape, lambda i: (0, 0)),
                     pl.BlockSpec((1, Dout), lambda i: (0, 0))]
        args += [head[0], head[1]]
    out_specs = [pl.BlockSpec((BE, Dout), lambda i: (i, 0))]
    out_shape = [jax.ShapeDtypeStruct((E, Dout), F32)]
    if do_stats:
        out_specs.append(pl.BlockSpec((2, Dout), lambda i: (0, 0)))
        out_shape.append(jax.ShapeDtypeStruct((2, Dout), F32))
    r = pl.pallas_call(
        body, grid=(T,),
        in_specs=in_specs,
        out_specs=out_specs,
        out_shape=tuple(out_shape))(*args)
    return r if do_stats else r[0]


def _bn_coefs(csum, csq, g, b, n):
    mu = csum / n
    var = csq / n - mu * mu
    s = g * lax.rsqrt(var + EPS)
    t = b - mu * s
    return s[None, :], t[None, :]


def kernel(node_features, edge_indices, edge_features, xbatch, params):
    p = params
    src = edge_indices[0].astype(jnp.int32)
    dst = edge_indices[1].astype(jnp.int32)
    src2 = src.reshape(NWIN, GW)
    dst2 = dst.reshape(NWIN, GW)
    zrows = jnp.zeros((NPAD // 16, NO), F32)

    cs_p, cd_p = _sc_counts(src, dst)
    cnt2 = _cnt_sum(cs_p, cd_p)[:, :N].T
    estats = _colstats(edge_features)

    e = edge_features
    n0stats = _colstats(node_features, rows=BN)
    xp, hx, es, xstats = _bn_project(
        node_features, n0stats, p['bn_node_g0'][None], p['bn_node_b0'][None],
        p['gat_W0'], p['gat_asrc0'][:, None], p['gat_adst0'][:, None], cnt2,
        want_gat=True)

    for i in range(3):
        ni = node_features.shape[1] if i == 0 else NO
        nei = edge_features.shape[1] if i == 0 else NO
        last = i == 2

        csum = jnp.concatenate([xstats[0], xstats[2], estats[0]])
        csq = jnp.concatenate([xstats[1], xstats[3], estats[1]])
        sc1, sh1 = _bn_coefs(csum, csq, p[f'e_bn1_g{i}'], p[f'e_bn1_b{i}'], E)

        xs, xd = _sc_gather(xp, src2, dst2)
        r = _edge_mlp1(xs, xd, e, sc1, sh1, p[f'e_lin1_W{i}'],
                       p[f'e_lin1_b{i}'][None], p[f'gat_W{i}'],
                       p[f'gat_asrc{i}'][:, None], p[f'gat_adst{i}'][:, None],
                       ni, nei, want_gat=not last)
        if last:
            h1, h1st = r
        else:
            h1, m, ex, h1st = r

        sc2, sh2 = _bn_coefs(h1st[0], h1st[1], p[f'e_bn2_g{i}'],
                             p[f'e_bn2_b{i}'], E)
        h2, h2st = _edge_mm(h1, sc2, sh2, p[f'e_lin2_W{i}'],
                            p[f'e_lin2_b{i}'][None], True, True)

        sc3, sh3 = _bn_coefs(h2st[0], h2st[1], p[f'e_bn3_g{i}'],
                             p[f'e_bn3_b{i}'], E)
        if last:
            return _edge_mm(h2, sc3, sh3, p[f'e_lin3_W{i}'],
                            p[f'e_lin3_b{i}'][None], False, False,
                            head=(p['pred_W'], p['pred_b'][None, :]))

        e, estats = _edge_mm(h2, sc3, sh3, p[f'e_lin3_W{i}'],
                             p[f'e_lin3_b{i}'][None], False, True)

        part = _sc_scatter_rows(m, dst2, zrows)
        denp = _sc_scatter_den(ex.reshape(E), dst)
        x, nstats = _gat_reduce(part, denp.T, hx, es,
                                p[f'gat_bias{i}'][None])

        r = _bn_project(x, nstats, p[f'bn_node_g{i + 1}'][None],
                        p[f'bn_node_b{i + 1}'][None], p[f'gat_W{i + 1}'],
                        p[f'gat_asrc{i + 1}'][:, None],
                        p[f'gat_adst{i + 1}'][:, None], cnt2,
                        want_gat=(i + 1 < 2))
        if i + 1 < 2:
            xp, hx, es, xstats = r
        else:
            xp, xstats = r


# speedup vs baseline: 1.9999x; 1.9999x over previous
import dataclasses
import functools

import jax
import jax.numpy as jnp
from jax import lax
from jax.experimental import pallas as pl
from jax.experimental.pallas import tpu as pltpu
from jax.experimental.pallas import tpu_sc as plsc

F32 = jnp.float32
N = 50000
E = 800000
NO = 32
LEAK = 0.1
EPS = 1e-5

NPAD = 50048
NW = 32
GW = 125
NWIN = E // GW
WPW = NWIN // NW
EPW = E // NW
CH = 1000
NCH = EPW // CH
BE = 4000
BN = 2000
CNT_CB = 2944


def _mesh():
    return plsc.VectorSubcoreMesh(core_axis_name="core", subcore_axis_name="subcore",
                                  num_cores=2, num_subcores=16)


def _sc_params():
    cp = pltpu.CompilerParams()
    fields = pltpu.CompilerParams.__dataclass_fields__
    if "needs_layout_passes" in fields:
        cp = dataclasses.replace(cp, needs_layout_passes=False)
    if "use_tc_tiling_on_sc" in fields:
        cp = dataclasses.replace(cp, use_tc_tiling_on_sc=False)
    return cp


def _leaky(x, s):
    return jnp.where(x >= 0, x, s * x)


def _sc_counts(src, dst):
    out_t = (jax.ShapeDtypeStruct((NW, NPAD), F32),
             jax.ShapeDtypeStruct((NW, NPAD), F32))

    @functools.partial(
        pl.kernel, out_type=out_t, mesh=_mesh(), compiler_params=_sc_params(),
        scratch_types=[pltpu.VMEM((NPAD,), F32), pltpu.VMEM((NPAD,), F32),
                       pltpu.VMEM((CH,), jnp.int32), pltpu.VMEM((CH,), jnp.int32)])
    def k(src_h, dst_h, cs_h, cd_h, cs_v, cd_v, s_v, d_v):
        wid = lax.axis_index("subcore") * 2 + lax.axis_index("core")
        zero = jnp.zeros((16,), F32)

        @pl.loop(0, NPAD, step=16)
        def _(i):
            cs_v[pl.ds(i, 16)] = zero
            cd_v[pl.ds(i, 16)] = zero

        base = wid * EPW
        ones = jnp.ones((16,), F32)
        tail = jnp.where(lax.iota(jnp.int32, 16) >= 8, 1.0, 0.0).astype(F32)

        @pl.loop(0, NCH)
        def _(j):
            pltpu.sync_copy(src_h.at[pl.ds(base + j * CH, CH)], s_v)
            pltpu.sync_copy(dst_h.at[pl.ds(base + j * CH, CH)], d_v)

            @pl.loop(0, CH - 16, step=16)
            def _(i):
                plsc.addupdate_scatter(cs_v, [s_v[pl.ds(i, 16)]], ones)
                plsc.addupdate_scatter(cd_v, [d_v[pl.ds(i, 16)]], ones)

            plsc.addupdate_scatter(cs_v, [s_v[pl.ds(CH - 16, 16)]], tail)
            plsc.addupdate_scatter(cd_v, [d_v[pl.ds(CH - 16, 16)]], tail)

        pltpu.sync_copy(cs_v, cs_h.at[wid])
        pltpu.sync_copy(cd_v, cd_h.at[wid])

    return k(src, dst)


def _sc_gather(xp, src2, dst2):
    D = xp.shape[1]
    out_t = (jax.ShapeDtypeStruct((E, D), F32), jax.ShapeDtypeStruct((E, D), F32))

    @functools.partial(pl.kernel, out_type=out_t, mesh=_mesh(),
                       compiler_params=_sc_params())
    def k(x_h, s_h, d_h, xs_h, xd_h):
        def body(s_v, d_v, xs_v, xd_v):
            pltpu.sync_copy(x_h.at[s_v.at[0]], xs_v)
            pltpu.sync_copy(x_h.at[d_v.at[0]], xd_v)

        pltpu.emit_pipeline(
            body, grid=(NWIN,),
            in_specs=[pl.BlockSpec((1, GW), lambda i: (i, 0)),
                      pl.BlockSpec((1, GW), lambda i: (i, 0))],
            out_specs=[pl.BlockSpec((GW, D), lambda i: (i, 0)),
                       pl.BlockSpec((GW, D), lambda i: (i, 0))],
            core_axis_name=("core", "subcore"),
            dimension_semantics=(pltpu.PARALLEL,),
        )(s_h, d_h, xs_h, xd_h)

    return k(xp, src2, dst2)


def _sc_scatter_rows(m, dst2, zrows):
    out_t = jax.ShapeDtypeStruct((2, NPAD, NO), F32)
    SCH = 500
    SNCH = EPW // SCH

    @functools.partial(
        pl.kernel, out_type=out_t, mesh=_mesh(), compiler_params=_sc_params(),
        scratch_types=[pltpu.VMEM_SHARED((NPAD, NO), F32),
                       pltpu.VMEM((SCH, NO), F32),
                       pltpu.VMEM((4, GW), jnp.int32)])
    def k(m_h, d2_h, z_h, part_h, acc, m_v, i2_v):
        cid = lax.axis_index("core")
        sid = lax.axis_index("subcore")
        wid = sid * 2 + cid
        rps = NPAD // 16

        pltpu.sync_copy(z_h, acc.at[pl.ds(sid * rps, rps)])
        plsc.subcore_barrier()

        base = wid * EPW
        rbase = wid * WPW

        @pl.loop(0, SNCH)
        def _(j):
            pltpu.sync_copy(m_h.at[pl.ds(base + j * SCH, SCH)], m_v)
            pltpu.sync_copy(d2_h.at[pl.ds(rbase + j * 4, 4)], i2_v)
            for r in range(4):
                pltpu.sync_copy(m_v.at[pl.ds(r * GW, GW)], acc.at[i2_v.at[r]],
                                add=True)

        plsc.subcore_barrier()
        pltpu.sync_copy(acc.at[pl.ds(sid * rps, rps)],
                        part_h.at[cid, pl.ds(sid * rps, rps)])

    return k(m, dst2, zrows)


def _sc_scatter_den(ex, dst1):
    out_t = jax.ShapeDtypeStruct((NW, NPAD), F32)

    @functools.partial(
        pl.kernel, out_type=out_t, mesh=_mesh(), compiler_params=_sc_params(),
        scratch_types=[pltpu.VMEM((NPAD,), F32),
                       pltpu.VMEM((CH,), jnp.int32),
                       pltpu.VMEM((CH,), F32)])
    def k(ex_h, d1_h, denp_h, den_v, d_v, e_v):
        wid = lax.axis_index("subcore") * 2 + lax.axis_index("core")
        zero = jnp.zeros((16,), F32)

        @pl.loop(0, NPAD, step=16)
        def _(i):
            den_v[pl.ds(i, 16)] = zero

        base = wid * EPW
        iota = lax.iota(jnp.int32, 16)

        @pl.loop(0, NCH)
        def _(j):
            eoff = base + j * CH
            pltpu.sync_copy(d1_h.at[pl.ds(eoff, CH)], d_v)
            pltpu.sync_copy(ex_h.at[pl.ds(eoff, CH)], e_v)

            @pl.loop(0, CH - 16, step=16)
            def _(i):
                plsc.addupdate_scatter(den_v, [d_v[pl.ds(i, 16)]],
                                       e_v[pl.ds(i, 16)])

            tv = jnp.where(iota >= 8, e_v[pl.ds(CH - 16, 16)], 0.0)
            plsc.addupdate_scatter(den_v, [d_v[pl.ds(CH - 16, 16)]], tv)

        pltpu.sync_copy(den_v, denp_h.at[wid])

    return k(ex, dst1)


def _colstats(x, rows=None):
    rows = BE if rows is None else rows
    D = x.shape[1]
    T = x.shape[0] // rows

    def body(x_ref, o_ref):
        @pl.when(pl.program_id(0) == 0)
        def _():
            o_ref[...] = jnp.zeros_like(o_ref)

        xv = x_ref[...]
        o_ref[...] += jnp.concatenate(
            [jnp.sum(xv, 0, keepdims=True), jnp.sum(xv * xv, 0, keepdims=True)], 0)

    return pl.pallas_call(
        body, grid=(T,),
        in_specs=[pl.BlockSpec((rows, D), lambda i: (i, 0))],
        out_specs=pl.BlockSpec((2, D), lambda i: (0, 0)),
        out_shape=jax.ShapeDtypeStruct((2, D), F32))(x)


def _cnt_sum(cs_p, cd_p):
    CB = CNT_CB
    T = NPAD // CB

    def body(cs_ref, cd_ref, o_ref):
        o_ref[...] = jnp.concatenate(
            [jnp.sum(cs_ref[...], 0, keepdims=True),
             jnp.sum(cd_ref[...], 0, keepdims=True)], 0)

    return pl.pallas_call(
        body, grid=(T,),
        in_specs=[pl.BlockSpec((NW, CB), lambda i: (0, i)),
                  pl.BlockSpec((NW, CB), lambda i: (0, i))],
        out_specs=pl.BlockSpec((2, CB), lambda i: (0, i)),
        out_shape=jax.ShapeDtypeStruct((2, NPAD), F32))(cs_p, cd_p)


def _bn_project(x, nstats, g, b, Wg, asr, adr, cnt2, want_gat):
    D = x.shape[1]
    T = N // BN

    def body(x_ref, ns_ref, g_ref, b_ref, w_ref, as_ref, ad_ref, c_ref, *outs):
        xv = x_ref[...]
        mu = ns_ref[0:1] * (1.0 / N)
        var = ns_ref[1:2] * (1.0 / N) - mu * mu
        xp = g_ref[...] * (xv - mu) * lax.rsqrt(var + EPS) + b_ref[...]
        cs = c_ref[...][:, 0:1]
        cd = c_ref[...][:, 1:2]
        xp2 = xp * xp
        xstats = jnp.concatenate([
            jnp.sum(xp * cs, 0, keepdims=True),
            jnp.sum(xp2 * cs, 0, keepdims=True),
            jnp.sum(xp * cd, 0, keepdims=True),
            jnp.sum(xp2 * cd, 0, keepdims=True)], 0)
        if want_gat:
            xp_ref, hx_ref, es_ref, st_ref = outs
            hx = jnp.dot(xp, w_ref[...], preferred_element_type=F32)
            a = (jnp.dot(hx, as_ref[...], preferred_element_type=F32)
                 + jnp.dot(hx, ad_ref[...], preferred_element_type=F32))
            hx_ref[...] = hx
            es_ref[...] = jnp.exp(_leaky(a, 0.2))
        else:
            xp_ref, st_ref = outs
        xp_ref[...] = xp

        @pl.when(pl.program_id(0) == 0)
        def _():
            st_ref[...] = jnp.zeros_like(st_ref)

        st_ref[...] += xstats

    in_specs = [pl.BlockSpec((BN, D), lambda i: (i, 0)),
                pl.BlockSpec((2, D), lambda i: (0, 0)),
                pl.BlockSpec((1, D), lambda i: (0, 0)),
                pl.BlockSpec((1, D), lambda i: (0, 0)),
                pl.BlockSpec((D, NO), lambda i: (0, 0)),
                pl.BlockSpec((NO, 1), lambda i: (0, 0)),
                pl.BlockSpec((NO, 1), lambda i: (0, 0)),
                pl.BlockSpec((BN, 2), lambda i: (i, 0))]
    if want_gat:
        out_specs = [pl.BlockSpec((BN, D), lambda i: (i, 0)),
                     pl.BlockSpec((BN, NO), lambda i: (i, 0)),
                     pl.BlockSpec((BN, 1), lambda i: (i, 0)),
                     pl.BlockSpec((4, D), lambda i: (0, 0))]
        out_shape = (jax.ShapeDtypeStruct((N, D), F32),
                     jax.ShapeDtypeStruct((N, NO), F32),
                     jax.ShapeDtypeStruct((N, 1), F32),
                     jax.ShapeDtypeStruct((4, D), F32))
    else:
        out_specs = [pl.BlockSpec((BN, D), lambda i: (i, 0)),
                     pl.BlockSpec((4, D), lambda i: (0, 0))]
        out_shape = (jax.ShapeDtypeStruct((N, D), F32),
                     jax.ShapeDtypeStruct((4, D), F32))
    return pl.pallas_call(
        body, grid=(T,), in_specs=in_specs, out_specs=out_specs,
        out_shape=out_shape)(x, nstats, g, b, Wg, asr, adr, cnt2)


def _gat_reduce(part, denpT, hx_prev, es_prev, bias):
    T = N // BN

    def body(p_ref, d_ref, hx_ref, es_ref, b_ref, x_ref, st_ref):
        es = es_ref[...]
        num = p_ref[0] + p_ref[1] + es * hx_ref[...]
        den = jnp.sum(d_ref[...], 1, keepdims=True) + es + 1e-16
        xv = _leaky(num / den + b_ref[...], LEAK)
        x_ref[...] = xv

        @pl.when(pl.program_id(0) == 0)
        def _():
            st_ref[...] = jnp.zeros_like(st_ref)

        st_ref[...] += jnp.concatenate(
            [jnp.sum(xv, 0, keepdims=True), jnp.sum(xv * xv, 0, keepdims=True)], 0)

    return pl.pallas_call(
        body, grid=(T,),
        in_specs=[pl.BlockSpec((2, BN, NO), lambda i: (0, i, 0)),
                  pl.BlockSpec((BN, NW), lambda i: (i, 0)),
                  pl.BlockSpec((BN, NO), lambda i: (i, 0)),
                  pl.BlockSpec((BN, 1), lambda i: (i, 0)),
                  pl.BlockSpec((1, NO), lambda i: (0, 0))],
        out_specs=[pl.BlockSpec((BN, NO), lambda i: (i, 0)),
                   pl.BlockSpec((2, NO), lambda i: (0, 0))],
        out_shape=(jax.ShapeDtypeStruct((N, NO), F32),
                   jax.ShapeDtypeStruct((2, NO), F32)))(
        part, denpT, hx_prev, es_prev, bias)


def _edge_mlp1(xs, xd, e, sc3, sh3, W1, b1, Wg, asr, adr, ni, nei, want_gat):
    T = E // BE
    din = 2 * ni + nei

    def body(xs_ref, xd_ref, e_ref, sc_ref, sh_ref, w1_ref, b1_ref, wg_ref,
             as_ref, ad_ref, *outs):
        xsv = xs_ref[...]
        xdv = xd_ref[...]
        ev = e_ref[...]
        sc = sc_ref[...]
        sh = sh_ref[...]
        W1 = w1_ref[...]
        xsn = xsv * sc[:, :ni] + sh[:, :ni]
        xdn = xdv * sc[:, ni:2 * ni] + sh[:, ni:2 * ni]
        en = ev * sc[:, 2 * ni:] + sh[:, 2 * ni:]
        z = (jnp.dot(xsn, W1[:ni], preferred_element_type=F32)
             + jnp.dot(xdn, W1[ni:2 * ni], preferred_element_type=F32)
             + jnp.dot(en, W1[2 * ni:], preferred_element_type=F32)
             + b1_ref[...])
        h1 = _leaky(z, LEAK)
        if want_gat:
            h1_ref, m_ref, ex_ref, st_ref = outs
            hxs = jnp.dot(xsv, wg_ref[...], preferred_element_type=F32)
            hxd = jnp.dot(xdv, wg_ref[...], preferred_element_type=F32)
            a = (jnp.dot(hxs, as_ref[...], preferred_element_type=F32)
                 + jnp.dot(hxd, ad_ref[...], preferred_element_type=F32))
            exv = jnp.exp(_leaky(a, 0.2))
            m_ref[...] = hxs * exv
            ex_ref[...] = exv
        else:
            h1_ref, st_ref = outs
        h1_ref[...] = h1

        @pl.when(pl.program_id(0) == 0)
        def _():
            st_ref[...] = jnp.zeros_like(st_ref)

        st_ref[...] += jnp.concatenate(
            [jnp.sum(h1, 0, keepdims=True), jnp.sum(h1 * h1, 0, keepdims=True)], 0)

    in_specs = [pl.BlockSpec((BE, ni), lambda i: (i, 0)),
                pl.BlockSpec((BE, ni), lambda i: (i, 0)),
                pl.BlockSpec((BE, nei), lambda i: (i, 0)),
                pl.BlockSpec((1, din), lambda i: (0, 0)),
                pl.BlockSpec((1, din), lambda i: (0, 0)),
                pl.BlockSpec((din, NO), lambda i: (0, 0)),
                pl.BlockSpec((1, NO), lambda i: (0, 0)),
                pl.BlockSpec((ni, NO), lambda i: (0, 0)),
                pl.BlockSpec((NO, 1), lambda i: (0, 0)),
                pl.BlockSpec((NO, 1), lambda i: (0, 0))]
    if want_gat:
        out_specs = [pl.BlockSpec((BE, NO), lambda i: (i, 0)),
                     pl.BlockSpec((BE, NO), lambda i: (i, 0)),
                     pl.BlockSpec((BE, 1), lambda i: (i, 0)),
                     pl.BlockSpec((2, NO), lambda i: (0, 0))]
        out_shape = (jax.ShapeDtypeStruct((E, NO), F32),
                     jax.ShapeDtypeStruct((E, NO), F32),
                     jax.ShapeDtypeStruct((E, 1), F32),
                     jax.ShapeDtypeStruct((2, NO), F32))
    else:
        out_specs = [pl.BlockSpec((BE, NO), lambda i: (i, 0)),
                     pl.BlockSpec((2, NO), lambda i: (0, 0))]
        out_shape = (jax.ShapeDtypeStruct((E, NO), F32),
                     jax.ShapeDtypeStruct((2, NO), F32))
    return pl.pallas_call(
        body, grid=(T,), in_specs=in_specs, out_specs=out_specs,
        out_shape=out_shape)(xs, xd, e, sc3, sh3, W1, b1, Wg, asr, adr)


def _edge_mm(h, sc, sh, W, b, do_leaky, do_stats, head=None):
    T = E // BE
    Din = h.shape[1]
    Dout = W.shape[1] if head is None else head[0].shape[1]

    def body(h_ref, sc_ref, sh_ref, w_ref, b_ref, *rest):
        if head is None:
            o_ref = rest[0]
            st = rest[1:]
        else:
            wp_ref, bp_ref, o_ref = rest[0], rest[1], rest[2]
            st = rest[3:]
        hn = h_ref[...] * sc_ref[...] + sh_ref[...]
        z = jnp.dot(hn, w_ref[...], preferred_element_type=F32) + b_ref[...]
        if do_leaky:
            z = _leaky(z, LEAK)
        if head is not None:
            z = jnp.dot(z, wp_ref[...], preferred_element_type=F32) + bp_ref[...]
        o_ref[...] = z
        if do_stats:
            st_ref = st[0]

            @pl.when(pl.program_id(0) == 0)
            def _():
                st_ref[...] = jnp.zeros_like(st_ref)

            st_ref[...] += jnp.concatenate(
                [jnp.sum(z, 0, keepdims=True), jnp.sum(z * z, 0, keepdims=True)], 0)

    in_specs = [pl.BlockSpec((BE, Din), lambda i: (i, 0)),
                pl.BlockSpec((1, Din), lambda i: (0, 0)),
                pl.BlockSpec((1, Din), lambda i: (0, 0)),
                pl.BlockSpec((Din, W.shape[1]), lambda i: (0, 0)),
                pl.BlockSpec((1, W.shape[1]), lambda i: (0, 0))]
    args = [h, sc, sh, W, b]
    if head is not None:
        in_specs += [pl.BlockSpec(head[0].shape, lambda i: (0, 0)),
                     pl.BlockSpec((1, Dout), lambda i: (0, 0))]
        args += [head[0], head[1]]
    out_specs = [pl.BlockSpec((BE, Dout), lambda i: (i, 0))]
    out_shape = [jax.ShapeDtypeStruct((E, Dout), F32)]
    if do_stats:
        out_specs.append(pl.BlockSpec((2, Dout), lambda i: (0, 0)))
        out_shape.append(jax.ShapeDtypeStruct((2, Dout), F32))
    r = pl.pallas_call(
        body, grid=(T,),
        in_specs=in_specs,
        out_specs=out_specs,
        out_shape=tuple(out_shape))(*args)
    return r if do_stats else r[0]


def _bn_coefs(csum, csq, g, b, n):
    mu = csum / n
    var = csq / n - mu * mu
    s = g * lax.rsqrt(var + EPS)
    t = b - mu * s
    return s[None, :], t[None, :]


def kernel(node_features, edge_indices, edge_features, xbatch, params):
    p = params
    src = edge_indices[0].astype(jnp.int32)
    dst = edge_indices[1].astype(jnp.int32)
    src2 = src.reshape(NWIN, GW)
    dst2 = dst.reshape(NWIN, GW)
    zrows = jnp.zeros((NPAD // 16, NO), F32)

    cs_p, cd_p = _sc_counts(src, dst)
    cnt2 = _cnt_sum(cs_p, cd_p)[:, :N].T
    estats = _colstats(edge_features)

    e = edge_features
    n0stats = _colstats(node_features, rows=BN)
    xp, hx, es, xstats = _bn_project(
        node_features, n0stats, p['bn_node_g0'][None], p['bn_node_b0'][None],
        p['gat_W0'], p['gat_asrc0'][:, None], p['gat_adst0'][:, None], cnt2,
        want_gat=True)

    for i in range(3):
        ni = node_features.shape[1] if i == 0 else NO
        nei = edge_features.shape[1] if i == 0 else NO
        last = i == 2

        csum = jnp.concatenate([xstats[0], xstats[2], estats[0]])
        csq = jnp.concatenate([xstats[1], xstats[3], estats[1]])
        sc1, sh1 = _bn_coefs(csum, csq, p[f'e_bn1_g{i}'], p[f'e_bn1_b{i}'], E)

        xs, xd = _sc_gather(xp, src2, dst2)
        r = _edge_mlp1(xs, xd, e, sc1, sh1, p[f'e_lin1_W{i}'],
                       p[f'e_lin1_b{i}'][None], p[f'gat_W{i}'],
                       p[f'gat_asrc{i}'][:, None], p[f'gat_adst{i}'][:, None],
                       ni, nei, want_gat=not last)
        if last:
            h1, h1st = r
        else:
            h1, m, ex, h1st = r

        sc2, sh2 = _bn_coefs(h1st[0], h1st[1], p[f'e_bn2_g{i}'],
                             p[f'e_bn2_b{i}'], E)
        h2, h2st = _edge_mm(h1, sc2, sh2, p[f'e_lin2_W{i}'],
                            p[f'e_lin2_b{i}'][None], True, True)

        sc3, sh3 = _bn_coefs(h2st[0], h2st[1], p[f'e_bn3_g{i}'],
                             p[f'e_bn3_b{i}'], E)
        if last:
            return _edge_mm(h2, sc3, sh3, p[f'e_lin3_W{i}'],
                            p[f'e_lin3_b{i}'][None], False, False,
                            head=(p['pred_W'], p['pred_b'][None, :]))

        e, estats = _edge_mm(h2, sc3, sh3, p[f'e_lin3_W{i}'],
                             p[f'e_lin3_b{i}'][None], False, True)

        part = _sc_scatter_rows(m, dst2, zrows)
        denp = _sc_scatter_den(ex.reshape(E), dst)
        x, nstats = _gat_reduce(part, denp.T, hx, es,
                                p[f'gat_bias{i}'][None])

        r = _bn_project(x, nstats, p[f'bn_node_g{i + 1}'][None],
                        p[f'bn_node_b{i + 1}'][None], p[f'gat_W{i + 1}'],
                        p[f'gat_asrc{i + 1}'][:, None],
                        p[f'gat_adst{i + 1}'][:, None], cnt2,
                        want_gat=(i + 1 < 2))
        if i + 1 < 2:
            xp, hx, es, xstats = r
        else:
            xp, xstats = r


# speedup vs baseline: 2.0177x; 1.0089x over previous
import dataclasses
import functools

import jax
import jax.numpy as jnp
from jax import lax
from jax.experimental import pallas as pl
from jax.experimental.pallas import tpu as pltpu
from jax.experimental.pallas import tpu_sc as plsc

F32 = jnp.float32
N = 50000
E = 800000
NO = 32
LEAK = 0.1
EPS = 1e-5

NPAD = 50048
NW = 32
GW = 125
NWIN = E // GW
WPW = NWIN // NW
EPW = E // NW
CH = 1000
NCH = EPW // CH
BE = 4000
BN = 2000
CNT_CB = 2944


def _mesh():
    return plsc.VectorSubcoreMesh(core_axis_name="core", subcore_axis_name="subcore",
                                  num_cores=2, num_subcores=16)


def _sc_params():
    cp = pltpu.CompilerParams()
    fields = pltpu.CompilerParams.__dataclass_fields__
    if "needs_layout_passes" in fields:
        cp = dataclasses.replace(cp, needs_layout_passes=False)
    if "use_tc_tiling_on_sc" in fields:
        cp = dataclasses.replace(cp, use_tc_tiling_on_sc=False)
    return cp


def _leaky(x, s):
    return jnp.where(x >= 0, x, s * x)


def _sc_counts(src, dst):
    out_t = (jax.ShapeDtypeStruct((NW, NPAD), F32),
             jax.ShapeDtypeStruct((NW, NPAD), F32))

    @functools.partial(
        pl.kernel, out_type=out_t, mesh=_mesh(), compiler_params=_sc_params(),
        scratch_types=[pltpu.VMEM((NPAD,), F32), pltpu.VMEM((NPAD,), F32),
                       pltpu.VMEM((CH,), jnp.int32), pltpu.VMEM((CH,), jnp.int32)])
    def k(src_h, dst_h, cs_h, cd_h, cs_v, cd_v, s_v, d_v):
        wid = lax.axis_index("subcore") * 2 + lax.axis_index("core")
        zero = jnp.zeros((16,), F32)

        @pl.loop(0, NPAD, step=16)
        def _(i):
            cs_v[pl.ds(i, 16)] = zero
            cd_v[pl.ds(i, 16)] = zero

        base = wid * EPW
        ones = jnp.ones((16,), F32)
        tail = jnp.where(lax.iota(jnp.int32, 16) >= 8, 1.0, 0.0).astype(F32)

        @pl.loop(0, NCH)
        def _(j):
            pltpu.sync_copy(src_h.at[pl.ds(base + j * CH, CH)], s_v)
            pltpu.sync_copy(dst_h.at[pl.ds(base + j * CH, CH)], d_v)

            @pl.loop(0, CH - 16, step=16)
            def _(i):
                plsc.addupdate_scatter(cs_v, [s_v[pl.ds(i, 16)]], ones)
                plsc.addupdate_scatter(cd_v, [d_v[pl.ds(i, 16)]], ones)

            plsc.addupdate_scatter(cs_v, [s_v[pl.ds(CH - 16, 16)]], tail)
            plsc.addupdate_scatter(cd_v, [d_v[pl.ds(CH - 16, 16)]], tail)

        pltpu.sync_copy(cs_v, cs_h.at[wid])
        pltpu.sync_copy(cd_v, cd_h.at[wid])

    return k(src, dst)


def _sc_gather(xp, src2, dst2):
    D = xp.shape[1]
    out_t = (jax.ShapeDtypeStruct((E, D), F32), jax.ShapeDtypeStruct((E, D), F32))

    @functools.partial(pl.kernel, out_type=out_t, mesh=_mesh(),
                       compiler_params=_sc_params(),
                       scratch_types=[pltpu.SemaphoreType.DMA])
    def k(x_h, s_h, d_h, xs_h, xd_h, sem):
        def body(s_v, d_v, xs_v, xd_v):
            c1 = pltpu.async_copy(x_h.at[s_v.at[0]], xs_v, sem)
            c2 = pltpu.async_copy(x_h.at[d_v.at[0]], xd_v, sem)
            c1.wait()
            c2.wait()

        pltpu.emit_pipeline(
            body, grid=(NWIN,),
            in_specs=[pl.BlockSpec((1, GW), lambda i: (i, 0)),
                      pl.BlockSpec((1, GW), lambda i: (i, 0))],
            out_specs=[pl.BlockSpec((GW, D), lambda i: (i, 0)),
                       pl.BlockSpec((GW, D), lambda i: (i, 0))],
            core_axis_name=("core", "subcore"),
            dimension_semantics=(pltpu.PARALLEL,),
        )(s_h, d_h, xs_h, xd_h)

    return k(xp, src2, dst2)


def _sc_scatter_rows(m, dst2, zrows):
    out_t = jax.ShapeDtypeStruct((2, NPAD, NO), F32)
    SCH = 500
    SNCH = EPW // SCH

    @functools.partial(
        pl.kernel, out_type=out_t, mesh=_mesh(), compiler_params=_sc_params(),
        scratch_types=[pltpu.VMEM_SHARED((NPAD, NO), F32),
                       pltpu.VMEM((SCH, NO), F32),
                       pltpu.VMEM((4, GW), jnp.int32)])
    def k(m_h, d2_h, z_h, part_h, acc, m_v, i2_v):
        cid = lax.axis_index("core")
        sid = lax.axis_index("subcore")
        wid = sid * 2 + cid
        rps = NPAD // 16

        pltpu.sync_copy(z_h, acc.at[pl.ds(sid * rps, rps)])
        plsc.subcore_barrier()

        base = wid * EPW
        rbase = wid * WPW

        @pl.loop(0, SNCH)
        def _(j):
            pltpu.sync_copy(m_h.at[pl.ds(base + j * SCH, SCH)], m_v)
            pltpu.sync_copy(d2_h.at[pl.ds(rbase + j * 4, 4)], i2_v)
            for r in range(4):
                pltpu.sync_copy(m_v.at[pl.ds(r * GW, GW)], acc.at[i2_v.at[r]],
                                add=True)

        plsc.subcore_barrier()
        pltpu.sync_copy(acc.at[pl.ds(sid * rps, rps)],
                        part_h.at[cid, pl.ds(sid * rps, rps)])

    return k(m, dst2, zrows)


def _sc_scatter_den(ex, dst1):
    out_t = jax.ShapeDtypeStruct((NW, NPAD), F32)

    @functools.partial(
        pl.kernel, out_type=out_t, mesh=_mesh(), compiler_params=_sc_params(),
        scratch_types=[pltpu.VMEM((NPAD,), F32),
                       pltpu.VMEM((CH,), jnp.int32),
                       pltpu.VMEM((CH,), F32)])
    def k(ex_h, d1_h, denp_h, den_v, d_v, e_v):
        wid = lax.axis_index("subcore") * 2 + lax.axis_index("core")
        zero = jnp.zeros((16,), F32)

        @pl.loop(0, NPAD, step=16)
        def _(i):
            den_v[pl.ds(i, 16)] = zero

        base = wid * EPW
        iota = lax.iota(jnp.int32, 16)

        @pl.loop(0, NCH)
        def _(j):
            eoff = base + j * CH
            pltpu.sync_copy(d1_h.at[pl.ds(eoff, CH)], d_v)
            pltpu.sync_copy(ex_h.at[pl.ds(eoff, CH)], e_v)

            @pl.loop(0, CH - 16, step=16)
            def _(i):
                plsc.addupdate_scatter(den_v, [d_v[pl.ds(i, 16)]],
                                       e_v[pl.ds(i, 16)])

            tv = jnp.where(iota >= 8, e_v[pl.ds(CH - 16, 16)], 0.0)
            plsc.addupdate_scatter(den_v, [d_v[pl.ds(CH - 16, 16)]], tv)

        pltpu.sync_copy(den_v, denp_h.at[wid])

    return k(ex, dst1)


def _colstats(x, rows=None):
    rows = BE if rows is None else rows
    D = x.shape[1]
    T = x.shape[0] // rows

    def body(x_ref, o_ref):
        @pl.when(pl.program_id(0) == 0)
        def _():
            o_ref[...] = jnp.zeros_like(o_ref)

        xv = x_ref[...]
        o_ref[...] += jnp.concatenate(
            [jnp.sum(xv, 0, keepdims=True), jnp.sum(xv * xv, 0, keepdims=True)], 0)

    return pl.pallas_call(
        body, grid=(T,),
        in_specs=[pl.BlockSpec((rows, D), lambda i: (i, 0))],
        out_specs=pl.BlockSpec((2, D), lambda i: (0, 0)),
        out_shape=jax.ShapeDtypeStruct((2, D), F32))(x)


def _cnt_sum(cs_p, cd_p):
    CB = CNT_CB
    T = NPAD // CB

    def body(cs_ref, cd_ref, o_ref):
        o_ref[...] = jnp.concatenate(
            [jnp.sum(cs_ref[...], 0, keepdims=True),
             jnp.sum(cd_ref[...], 0, keepdims=True)], 0)

    return pl.pallas_call(
        body, grid=(T,),
        in_specs=[pl.BlockSpec((NW, CB), lambda i: (0, i)),
                  pl.BlockSpec((NW, CB), lambda i: (0, i))],
        out_specs=pl.BlockSpec((2, CB), lambda i: (0, i)),
        out_shape=jax.ShapeDtypeStruct((2, NPAD), F32))(cs_p, cd_p)


def _bn_project(x, nstats, g, b, Wg, asr, adr, cnt2, want_gat):
    D = x.shape[1]
    T = N // BN

    def body(x_ref, ns_ref, g_ref, b_ref, w_ref, as_ref, ad_ref, c_ref, *outs):
        xv = x_ref[...]
        mu = ns_ref[0:1] * (1.0 / N)
        var = ns_ref[1:2] * (1.0 / N) - mu * mu
        xp = g_ref[...] * (xv - mu) * lax.rsqrt(var + EPS) + b_ref[...]
        cs = c_ref[...][:, 0:1]
        cd = c_ref[...][:, 1:2]
        xp2 = xp * xp
        xstats = jnp.concatenate([
            jnp.sum(xp * cs, 0, keepdims=True),
            jnp.sum(xp2 * cs, 0, keepdims=True),
            jnp.sum(xp * cd, 0, keepdims=True),
            jnp.sum(xp2 * cd, 0, keepdims=True)], 0)
        if want_gat:
            xp_ref, hx_ref, es_ref, st_ref = outs
            hx = jnp.dot(xp, w_ref[...], preferred_element_type=F32)
            a = (jnp.dot(hx, as_ref[...], preferred_element_type=F32)
                 + jnp.dot(hx, ad_ref[...], preferred_element_type=F32))
            hx_ref[...] = hx
            es_ref[...] = jnp.exp(_leaky(a, 0.2))
        else:
            xp_ref, st_ref = outs
        xp_ref[...] = xp

        @pl.when(pl.program_id(0) == 0)
        def _():
            st_ref[...] = jnp.zeros_like(st_ref)

        st_ref[...] += xstats

    in_specs = [pl.BlockSpec((BN, D), lambda i: (i, 0)),
                pl.BlockSpec((2, D), lambda i: (0, 0)),
                pl.BlockSpec((1, D), lambda i: (0, 0)),
                pl.BlockSpec((1, D), lambda i: (0, 0)),
                pl.BlockSpec((D, NO), lambda i: (0, 0)),
                pl.BlockSpec((NO, 1), lambda i: (0, 0)),
                pl.BlockSpec((NO, 1), lambda i: (0, 0)),
                pl.BlockSpec((BN, 2), lambda i: (i, 0))]
    if want_gat:
        out_specs = [pl.BlockSpec((BN, D), lambda i: (i, 0)),
                     pl.BlockSpec((BN, NO), lambda i: (i, 0)),
                     pl.BlockSpec((BN, 1), lambda i: (i, 0)),
                     pl.BlockSpec((4, D), lambda i: (0, 0))]
        out_shape = (jax.ShapeDtypeStruct((N, D), F32),
                     jax.ShapeDtypeStruct((N, NO), F32),
                     jax.ShapeDtypeStruct((N, 1), F32),
                     jax.ShapeDtypeStruct((4, D), F32))
    else:
        out_specs = [pl.BlockSpec((BN, D), lambda i: (i, 0)),
                     pl.BlockSpec((4, D), lambda i: (0, 0))]
        out_shape = (jax.ShapeDtypeStruct((N, D), F32),
                     jax.ShapeDtypeStruct((4, D), F32))
    return pl.pallas_call(
        body, grid=(T,), in_specs=in_specs, out_specs=out_specs,
        out_shape=out_shape)(x, nstats, g, b, Wg, asr, adr, cnt2)


def _gat_reduce(part, denpT, hx_prev, es_prev, bias):
    T = N // BN

    def body(p_ref, d_ref, hx_ref, es_ref, b_ref, x_ref, st_ref):
        es = es_ref[...]
        num = p_ref[0] + p_ref[1] + es * hx_ref[...]
        den = jnp.sum(d_ref[...], 1, keepdims=True) + es + 1e-16
        xv = _leaky(num / den + b_ref[...], LEAK)
        x_ref[...] = xv

        @pl.when(pl.program_id(0) == 0)
        def _():
            st_ref[...] = jnp.zeros_like(st_ref)

        st_ref[...] += jnp.concatenate(
            [jnp.sum(xv, 0, keepdims=True), jnp.sum(xv * xv, 0, keepdims=True)], 0)

    return pl.pallas_call(
        body, grid=(T,),
        in_specs=[pl.BlockSpec((2, BN, NO), lambda i: (0, i, 0)),
                  pl.BlockSpec((BN, NW), lambda i: (i, 0)),
                  pl.BlockSpec((BN, NO), lambda i: (i, 0)),
                  pl.BlockSpec((BN, 1), lambda i: (i, 0)),
                  pl.BlockSpec((1, NO), lambda i: (0, 0))],
        out_specs=[pl.BlockSpec((BN, NO), lambda i: (i, 0)),
                   pl.BlockSpec((2, NO), lambda i: (0, 0))],
        out_shape=(jax.ShapeDtypeStruct((N, NO), F32),
                   jax.ShapeDtypeStruct((2, NO), F32)))(
        part, denpT, hx_prev, es_prev, bias)


def _edge_mlp1(xs, xd, e, sc3, sh3, W1, b1, Wg, asr, adr, ni, nei, want_gat):
    T = E // BE
    din = 2 * ni + nei

    def body(xs_ref, xd_ref, e_ref, sc_ref, sh_ref, w1_ref, b1_ref, wg_ref,
             as_ref, ad_ref, *outs):
        xsv = xs_ref[...]
        xdv = xd_ref[...]
        ev = e_ref[...]
        sc = sc_ref[...]
        sh = sh_ref[...]
        W1 = w1_ref[...]
        xsn = xsv * sc[:, :ni] + sh[:, :ni]
        xdn = xdv * sc[:, ni:2 * ni] + sh[:, ni:2 * ni]
        en = ev * sc[:, 2 * ni:] + sh[:, 2 * ni:]
        z = (jnp.dot(xsn, W1[:ni], preferred_element_type=F32)
             + jnp.dot(xdn, W1[ni:2 * ni], preferred_element_type=F32)
             + jnp.dot(en, W1[2 * ni:], preferred_element_type=F32)
             + b1_ref[...])
        h1 = _leaky(z, LEAK)
        if want_gat:
            h1_ref, m_ref, ex_ref, st_ref = outs
            hxs = jnp.dot(xsv, wg_ref[...], preferred_element_type=F32)
            hxd = jnp.dot(xdv, wg_ref[...], preferred_element_type=F32)
            a = (jnp.dot(hxs, as_ref[...], preferred_element_type=F32)
                 + jnp.dot(hxd, ad_ref[...], preferred_element_type=F32))
            exv = jnp.exp(_leaky(a, 0.2))
            m_ref[...] = hxs * exv
            ex_ref[...] = exv
        else:
            h1_ref, st_ref = outs
        h1_ref[...] = h1

        @pl.when(pl.program_id(0) == 0)
        def _():
            st_ref[...] = jnp.zeros_like(st_ref)

        st_ref[...] += jnp.concatenate(
            [jnp.sum(h1, 0, keepdims=True), jnp.sum(h1 * h1, 0, keepdims=True)], 0)

    in_specs = [pl.BlockSpec((BE, ni), lambda i: (i, 0)),
                pl.BlockSpec((BE, ni), lambda i: (i, 0)),
                pl.BlockSpec((BE, nei), lambda i: (i, 0)),
                pl.BlockSpec((1, din), lambda i: (0, 0)),
                pl.BlockSpec((1, din), lambda i: (0, 0)),
                pl.BlockSpec((din, NO), lambda i: (0, 0)),
                pl.BlockSpec((1, NO), lambda i: (0, 0)),
                pl.BlockSpec((ni, NO), lambda i: (0, 0)),
                pl.BlockSpec((NO, 1), lambda i: (0, 0)),
                pl.BlockSpec((NO, 1), lambda i: (0, 0))]
    if want_gat:
        out_specs = [pl.BlockSpec((BE, NO), lambda i: (i, 0)),
                     pl.BlockSpec((BE, NO), lambda i: (i, 0)),
                     pl.BlockSpec((BE, 1), lambda i: (i, 0)),
                     pl.BlockSpec((2, NO), lambda i: (0, 0))]
        out_shape = (jax.ShapeDtypeStruct((E, NO), F32),
                     jax.ShapeDtypeStruct((E, NO), F32),
                     jax.ShapeDtypeStruct((E, 1), F32),
                     jax.ShapeDtypeStruct((2, NO), F32))
    else:
        out_specs = [pl.BlockSpec((BE, NO), lambda i: (i, 0)),
                     pl.BlockSpec((2, NO), lambda i: (0, 0))]
        out_shape = (jax.ShapeDtypeStruct((E, NO), F32),
                     jax.ShapeDtypeStruct((2, NO), F32))
    return pl.pallas_call(
        body, grid=(T,), in_specs=in_specs, out_specs=out_specs,
        out_shape=out_shape)(xs, xd, e, sc3, sh3, W1, b1, Wg, asr, adr)


def _edge_mm(h, sc, sh, W, b, do_leaky, do_stats, head=None):
    T = E // BE
    Din = h.shape[1]
    Dout = W.shape[1] if head is None else head[0].shape[1]

    def body(h_ref, sc_ref, sh_ref, w_ref, b_ref, *rest):
        if head is None:
            o_ref = rest[0]
            st = rest[1:]
        else:
            wp_ref, bp_ref, o_ref = rest[0], rest[1], rest[2]
            st = rest[3:]
        hn = h_ref[...] * sc_ref[...] + sh_ref[...]
        z = jnp.dot(hn, w_ref[...], preferred_element_type=F32) + b_ref[...]
        if do_leaky:
            z = _leaky(z, LEAK)
        if head is not None:
            z = jnp.dot(z, wp_ref[...], preferred_element_type=F32) + bp_ref[...]
        o_ref[...] = z
        if do_stats:
            st_ref = st[0]

            @pl.when(pl.program_id(0) == 0)
            def _():
                st_ref[...] = jnp.zeros_like(st_ref)

            st_ref[...] += jnp.concatenate(
                [jnp.sum(z, 0, keepdims=True), jnp.sum(z * z, 0, keepdims=True)], 0)

    in_specs = [pl.BlockSpec((BE, Din), lambda i: (i, 0)),
                pl.BlockSpec((1, Din), lambda i: (0, 0)),
                pl.BlockSpec((1, Din), lambda i: (0, 0)),
                pl.BlockSpec((Din, W.shape[1]), lambda i: (0, 0)),
                pl.BlockSpec((1, W.shape[1]), lambda i: (0, 0))]
    args = [h, sc, sh, W, b]
    if head is not None:
        in_specs += [pl.BlockSpec(head[0].shape, lambda i: (0, 0)),
                     pl.BlockSpec((1, Dout), lambda i: (0, 0))]
        args += [head[0], head[1]]
    out_specs = [pl.BlockSpec((BE, Dout), lambda i: (i, 0))]
    out_shape = [jax.ShapeDtypeStruct((E, Dout), F32)]
    if do_stats:
        out_specs.append(pl.BlockSpec((2, Dout), lambda i: (0, 0)))
        out_shape.append(jax.ShapeDtypeStruct((2, Dout), F32))
    r = pl.pallas_call(
        body, grid=(T,),
        in_specs=in_specs,
        out_specs=out_specs,
        out_shape=tuple(out_shape))(*args)
    return r if do_stats else r[0]


def _bn_coefs(csum, csq, g, b, n):
    mu = csum / n
    var = csq / n - mu * mu
    s = g * lax.rsqrt(var + EPS)
    t = b - mu * s
    return s[None, :], t[None, :]


def kernel(node_features, edge_indices, edge_features, xbatch, params):
    p = params
    src = edge_indices[0].astype(jnp.int32)
    dst = edge_indices[1].astype(jnp.int32)
    src2 = src.reshape(NWIN, GW)
    dst2 = dst.reshape(NWIN, GW)
    zrows = jnp.zeros((NPAD // 16, NO), F32)

    cs_p, cd_p = _sc_counts(src, dst)
    cnt2 = _cnt_sum(cs_p, cd_p)[:, :N].T
    estats = _colstats(edge_features)

    e = edge_features
    n0stats = _colstats(node_features, rows=BN)
    xp, hx, es, xstats = _bn_project(
        node_features, n0stats, p['bn_node_g0'][None], p['bn_node_b0'][None],
        p['gat_W0'], p['gat_asrc0'][:, None], p['gat_adst0'][:, None], cnt2,
        want_gat=True)

    for i in range(3):
        ni = node_features.shape[1] if i == 0 else NO
        nei = edge_features.shape[1] if i == 0 else NO
        last = i == 2

        csum = jnp.concatenate([xstats[0], xstats[2], estats[0]])
        csq = jnp.concatenate([xstats[1], xstats[3], estats[1]])
        sc1, sh1 = _bn_coefs(csum, csq, p[f'e_bn1_g{i}'], p[f'e_bn1_b{i}'], E)

        xs, xd = _sc_gather(xp, src2, dst2)
        r = _edge_mlp1(xs, xd, e, sc1, sh1, p[f'e_lin1_W{i}'],
                       p[f'e_lin1_b{i}'][None], p[f'gat_W{i}'],
                       p[f'gat_asrc{i}'][:, None], p[f'gat_adst{i}'][:, None],
                       ni, nei, want_gat=not last)
        if last:
            h1, h1st = r
        else:
            h1, m, ex, h1st = r

        sc2, sh2 = _bn_coefs(h1st[0], h1st[1], p[f'e_bn2_g{i}'],
                             p[f'e_bn2_b{i}'], E)
        h2, h2st = _edge_mm(h1, sc2, sh2, p[f'e_lin2_W{i}'],
                            p[f'e_lin2_b{i}'][None], True, True)

        sc3, sh3 = _bn_coefs(h2st[0], h2st[1], p[f'e_bn3_g{i}'],
                             p[f'e_bn3_b{i}'], E)
        if last:
            return _edge_mm(h2, sc3, sh3, p[f'e_lin3_W{i}'],
                            p[f'e_lin3_b{i}'][None], False, False,
                            head=(p['pred_W'], p['pred_b'][None, :]))

        e, estats = _edge_mm(h2, sc3, sh3, p[f'e_lin3_W{i}'],
                             p[f'e_lin3_b{i}'][None], False, True)

        part = _sc_scatter_rows(m, dst2, zrows)
        denp = _sc_scatter_den(ex.reshape(E), dst)
        x, nstats = _gat_reduce(part, denp.T, hx, es,
                                p[f'gat_bias{i}'][None])

        r = _bn_project(x, nstats, p[f'bn_node_g{i + 1}'][None],
                        p[f'bn_node_b{i + 1}'][None], p[f'gat_W{i + 1}'],
                        p[f'gat_asrc{i + 1}'][:, None],
                        p[f'gat_adst{i + 1}'][:, None], cnt2,
                        want_gat=(i + 1 < 2))
        if i + 1 < 2:
            xp, hx, es, xstats = r
        else:
            xp, xstats = r


# speedup vs baseline: 2.1434x; 1.0623x over previous
import dataclasses
import functools

import jax
import jax.numpy as jnp
from jax import lax
from jax.experimental import pallas as pl
from jax.experimental.pallas import tpu as pltpu
from jax.experimental.pallas import tpu_sc as plsc

F32 = jnp.float32
N = 50000
E = 800000
NO = 32
LEAK = 0.1
EPS = 1e-5

NPAD = 50048
NW = 32
GW = 125
NWIN = E // GW
WPW = NWIN // NW
EPW = E // NW
CH = 1000
NCH = EPW // CH
BE = 8000
BN = 2000
CNT_CB = 2944


def _mesh():
    return plsc.VectorSubcoreMesh(core_axis_name="core", subcore_axis_name="subcore",
                                  num_cores=2, num_subcores=16)


def _sc_params():
    cp = pltpu.CompilerParams()
    fields = pltpu.CompilerParams.__dataclass_fields__
    if "needs_layout_passes" in fields:
        cp = dataclasses.replace(cp, needs_layout_passes=False)
    if "use_tc_tiling_on_sc" in fields:
        cp = dataclasses.replace(cp, use_tc_tiling_on_sc=False)
    return cp


def _leaky(x, s):
    return jnp.where(x >= 0, x, s * x)


def _sc_counts(src, dst):
    out_t = (jax.ShapeDtypeStruct((NW, NPAD), F32),
             jax.ShapeDtypeStruct((NW, NPAD), F32))

    @functools.partial(
        pl.kernel, out_type=out_t, mesh=_mesh(), compiler_params=_sc_params(),
        scratch_types=[pltpu.VMEM((NPAD,), F32), pltpu.VMEM((NPAD,), F32),
                       pltpu.VMEM((CH,), jnp.int32), pltpu.VMEM((CH,), jnp.int32)])
    def k(src_h, dst_h, cs_h, cd_h, cs_v, cd_v, s_v, d_v):
        wid = lax.axis_index("subcore") * 2 + lax.axis_index("core")
        zero = jnp.zeros((16,), F32)

        @pl.loop(0, NPAD, step=16)
        def _(i):
            cs_v[pl.ds(i, 16)] = zero
            cd_v[pl.ds(i, 16)] = zero

        base = wid * EPW
        ones = jnp.ones((16,), F32)
        tail = jnp.where(lax.iota(jnp.int32, 16) >= 8, 1.0, 0.0).astype(F32)

        @pl.loop(0, NCH)
        def _(j):
            pltpu.sync_copy(src_h.at[pl.ds(base + j * CH, CH)], s_v)
            pltpu.sync_copy(dst_h.at[pl.ds(base + j * CH, CH)], d_v)

            @pl.loop(0, CH - 16, step=16)
            def _(i):
                plsc.addupdate_scatter(cs_v, [s_v[pl.ds(i, 16)]], ones)
                plsc.addupdate_scatter(cd_v, [d_v[pl.ds(i, 16)]], ones)

            plsc.addupdate_scatter(cs_v, [s_v[pl.ds(CH - 16, 16)]], tail)
            plsc.addupdate_scatter(cd_v, [d_v[pl.ds(CH - 16, 16)]], tail)

        pltpu.sync_copy(cs_v, cs_h.at[wid])
        pltpu.sync_copy(cd_v, cd_h.at[wid])

    return k(src, dst)


def _sc_gather(xp, src2, dst2):
    D = xp.shape[1]
    out_t = (jax.ShapeDtypeStruct((E, D), F32), jax.ShapeDtypeStruct((E, D), F32))

    @functools.partial(pl.kernel, out_type=out_t, mesh=_mesh(),
                       compiler_params=_sc_params(),
                       scratch_types=[pltpu.SemaphoreType.DMA])
    def k(x_h, s_h, d_h, xs_h, xd_h, sem):
        def body(s_v, d_v, xs_v, xd_v):
            c1 = pltpu.async_copy(x_h.at[s_v.at[0]], xs_v, sem)
            c2 = pltpu.async_copy(x_h.at[d_v.at[0]], xd_v, sem)
            c1.wait()
            c2.wait()

        pltpu.emit_pipeline(
            body, grid=(NWIN,),
            in_specs=[pl.BlockSpec((1, GW), lambda i: (i, 0)),
                      pl.BlockSpec((1, GW), lambda i: (i, 0))],
            out_specs=[pl.BlockSpec((GW, D), lambda i: (i, 0)),
                       pl.BlockSpec((GW, D), lambda i: (i, 0))],
            core_axis_name=("core", "subcore"),
            dimension_semantics=(pltpu.PARALLEL,),
        )(s_h, d_h, xs_h, xd_h)

    return k(xp, src2, dst2)


def _sc_scatter_rows(m, dst2, zrows):
    out_t = jax.ShapeDtypeStruct((2, NPAD, NO), F32)
    SCH = 500
    SNCH = EPW // SCH

    @functools.partial(
        pl.kernel, out_type=out_t, mesh=_mesh(), compiler_params=_sc_params(),
        scratch_types=[pltpu.VMEM_SHARED((NPAD, NO), F32),
                       pltpu.VMEM((SCH, NO), F32),
                       pltpu.VMEM((4, GW), jnp.int32)])
    def k(m_h, d2_h, z_h, part_h, acc, m_v, i2_v):
        cid = lax.axis_index("core")
        sid = lax.axis_index("subcore")
        wid = sid * 2 + cid
        rps = NPAD // 16

        pltpu.sync_copy(z_h, acc.at[pl.ds(sid * rps, rps)])
        plsc.subcore_barrier()

        base = wid * EPW
        rbase = wid * WPW

        @pl.loop(0, SNCH)
        def _(j):
            pltpu.sync_copy(m_h.at[pl.ds(base + j * SCH, SCH)], m_v)
            pltpu.sync_copy(d2_h.at[pl.ds(rbase + j * 4, 4)], i2_v)
            for r in range(4):
                pltpu.sync_copy(m_v.at[pl.ds(r * GW, GW)], acc.at[i2_v.at[r]],
                                add=True)

        plsc.subcore_barrier()
        pltpu.sync_copy(acc.at[pl.ds(sid * rps, rps)],
                        part_h.at[cid, pl.ds(sid * rps, rps)])

    return k(m, dst2, zrows)


def _sc_scatter_den(ex, dst1):
    out_t = jax.ShapeDtypeStruct((NW, NPAD), F32)

    @functools.partial(
        pl.kernel, out_type=out_t, mesh=_mesh(), compiler_params=_sc_params(),
        scratch_types=[pltpu.VMEM((NPAD,), F32),
                       pltpu.VMEM((CH,), jnp.int32),
                       pltpu.VMEM((CH,), F32)])
    def k(ex_h, d1_h, denp_h, den_v, d_v, e_v):
        wid = lax.axis_index("subcore") * 2 + lax.axis_index("core")
        zero = jnp.zeros((16,), F32)

        @pl.loop(0, NPAD, step=16)
        def _(i):
            den_v[pl.ds(i, 16)] = zero

        base = wid * EPW
        iota = lax.iota(jnp.int32, 16)

        @pl.loop(0, NCH)
        def _(j):
            eoff = base + j * CH
            pltpu.sync_copy(d1_h.at[pl.ds(eoff, CH)], d_v)
            pltpu.sync_copy(ex_h.at[pl.ds(eoff, CH)], e_v)

            @pl.loop(0, CH - 16, step=16)
            def _(i):
                plsc.addupdate_scatter(den_v, [d_v[pl.ds(i, 16)]],
                                       e_v[pl.ds(i, 16)])

            tv = jnp.where(iota >= 8, e_v[pl.ds(CH - 16, 16)], 0.0)
            plsc.addupdate_scatter(den_v, [d_v[pl.ds(CH - 16, 16)]], tv)

        pltpu.sync_copy(den_v, denp_h.at[wid])

    return k(ex, dst1)


def _colstats(x, rows=None):
    rows = BE if rows is None else rows
    D = x.shape[1]
    T = x.shape[0] // rows

    def body(x_ref, o_ref):
        @pl.when(pl.program_id(0) == 0)
        def _():
            o_ref[...] = jnp.zeros_like(o_ref)

        xv = x_ref[...]
        o_ref[...] += jnp.concatenate(
            [jnp.sum(xv, 0, keepdims=True), jnp.sum(xv * xv, 0, keepdims=True)], 0)

    return pl.pallas_call(
        body, grid=(T,),
        in_specs=[pl.BlockSpec((rows, D), lambda i: (i, 0))],
        out_specs=pl.BlockSpec((2, D), lambda i: (0, 0)),
        out_shape=jax.ShapeDtypeStruct((2, D), F32))(x)


def _cnt_sum(cs_p, cd_p):
    CB = CNT_CB
    T = NPAD // CB

    def body(cs_ref, cd_ref, o_ref):
        o_ref[...] = jnp.concatenate(
            [jnp.sum(cs_ref[...], 0, keepdims=True),
             jnp.sum(cd_ref[...], 0, keepdims=True)], 0)

    return pl.pallas_call(
        body, grid=(T,),
        in_specs=[pl.BlockSpec((NW, CB), lambda i: (0, i)),
                  pl.BlockSpec((NW, CB), lambda i: (0, i))],
        out_specs=pl.BlockSpec((2, CB), lambda i: (0, i)),
        out_shape=jax.ShapeDtypeStruct((2, NPAD), F32))(cs_p, cd_p)


def _bn_project(x, nstats, g, b, Wg, asr, adr, cnt2, want_gat):
    D = x.shape[1]
    T = N // BN

    def body(x_ref, ns_ref, g_ref, b_ref, w_ref, as_ref, ad_ref, c_ref, *outs):
        xv = x_ref[...]
        mu = ns_ref[0:1] * (1.0 / N)
        var = ns_ref[1:2] * (1.0 / N) - mu * mu
        xp = g_ref[...] * (xv - mu) * lax.rsqrt(var + EPS) + b_ref[...]
        cs = c_ref[...][:, 0:1]
        cd = c_ref[...][:, 1:2]
        xp2 = xp * xp
        xstats = jnp.concatenate([
            jnp.sum(xp * cs, 0, keepdims=True),
            jnp.sum(xp2 * cs, 0, keepdims=True),
            jnp.sum(xp * cd, 0, keepdims=True),
            jnp.sum(xp2 * cd, 0, keepdims=True)], 0)
        if want_gat:
            xp_ref, hx_ref, es_ref, st_ref = outs
            hx = jnp.dot(xp, w_ref[...], preferred_element_type=F32)
            a = (jnp.dot(hx, as_ref[...], preferred_element_type=F32)
                 + jnp.dot(hx, ad_ref[...], preferred_element_type=F32))
            hx_ref[...] = hx
            es_ref[...] = jnp.exp(_leaky(a, 0.2))
        else:
            xp_ref, st_ref = outs
        xp_ref[...] = xp

        @pl.when(pl.program_id(0) == 0)
        def _():
            st_ref[...] = jnp.zeros_like(st_ref)

        st_ref[...] += xstats

    in_specs = [pl.BlockSpec((BN, D), lambda i: (i, 0)),
                pl.BlockSpec((2, D), lambda i: (0, 0)),
                pl.BlockSpec((1, D), lambda i: (0, 0)),
                pl.BlockSpec((1, D), lambda i: (0, 0)),
                pl.BlockSpec((D, NO), lambda i: (0, 0)),
                pl.BlockSpec((NO, 1), lambda i: (0, 0)),
                pl.BlockSpec((NO, 1), lambda i: (0, 0)),
                pl.BlockSpec((BN, 2), lambda i: (i, 0))]
    if want_gat:
        out_specs = [pl.BlockSpec((BN, D), lambda i: (i, 0)),
                     pl.BlockSpec((BN, NO), lambda i: (i, 0)),
                     pl.BlockSpec((BN, 1), lambda i: (i, 0)),
                     pl.BlockSpec((4, D), lambda i: (0, 0))]
        out_shape = (jax.ShapeDtypeStruct((N, D), F32),
                     jax.ShapeDtypeStruct((N, NO), F32),
                     jax.ShapeDtypeStruct((N, 1), F32),
                     jax.ShapeDtypeStruct((4, D), F32))
    else:
        out_specs = [pl.BlockSpec((BN, D), lambda i: (i, 0)),
                     pl.BlockSpec((4, D), lambda i: (0, 0))]
        out_shape = (jax.ShapeDtypeStruct((N, D), F32),
                     jax.ShapeDtypeStruct((4, D), F32))
    return pl.pallas_call(
        body, grid=(T,), in_specs=in_specs, out_specs=out_specs,
        out_shape=out_shape)(x, nstats, g, b, Wg, asr, adr, cnt2)


def _gat_reduce(part, denpT, hx_prev, es_prev, bias):
    T = N // BN

    def body(p_ref, d_ref, hx_ref, es_ref, b_ref, x_ref, st_ref):
        es = es_ref[...]
        num = p_ref[0] + p_ref[1] + es * hx_ref[...]
        den = jnp.sum(d_ref[...], 1, keepdims=True) + es + 1e-16
        xv = _leaky(num / den + b_ref[...], LEAK)
        x_ref[...] = xv

        @pl.when(pl.program_id(0) == 0)
        def _():
            st_ref[...] = jnp.zeros_like(st_ref)

        st_ref[...] += jnp.concatenate(
            [jnp.sum(xv, 0, keepdims=True), jnp.sum(xv * xv, 0, keepdims=True)], 0)

    return pl.pallas_call(
        body, grid=(T,),
        in_specs=[pl.BlockSpec((2, BN, NO), lambda i: (0, i, 0)),
                  pl.BlockSpec((BN, NW), lambda i: (i, 0)),
                  pl.BlockSpec((BN, NO), lambda i: (i, 0)),
                  pl.BlockSpec((BN, 1), lambda i: (i, 0)),
                  pl.BlockSpec((1, NO), lambda i: (0, 0))],
        out_specs=[pl.BlockSpec((BN, NO), lambda i: (i, 0)),
                   pl.BlockSpec((2, NO), lambda i: (0, 0))],
        out_shape=(jax.ShapeDtypeStruct((N, NO), F32),
                   jax.ShapeDtypeStruct((2, NO), F32)))(
        part, denpT, hx_prev, es_prev, bias)


def _edge_mlp1(xs, xd, e, sc3, sh3, W1, b1, Wg, asr, adr, ni, nei, want_gat):
    T = E // BE
    din = 2 * ni + nei

    def body(xs_ref, xd_ref, e_ref, sc_ref, sh_ref, w1_ref, b1_ref, wg_ref,
             as_ref, ad_ref, *outs):
        xsv = xs_ref[...]
        xdv = xd_ref[...]
        ev = e_ref[...]
        sc = sc_ref[...]
        sh = sh_ref[...]
        W1 = w1_ref[...]
        xsn = xsv * sc[:, :ni] + sh[:, :ni]
        xdn = xdv * sc[:, ni:2 * ni] + sh[:, ni:2 * ni]
        en = ev * sc[:, 2 * ni:] + sh[:, 2 * ni:]
        z = (jnp.dot(xsn, W1[:ni], preferred_element_type=F32)
             + jnp.dot(xdn, W1[ni:2 * ni], preferred_element_type=F32)
             + jnp.dot(en, W1[2 * ni:], preferred_element_type=F32)
             + b1_ref[...])
        h1 = _leaky(z, LEAK)
        if want_gat:
            h1_ref, m_ref, ex_ref, st_ref = outs
            hxs = jnp.dot(xsv, wg_ref[...], preferred_element_type=F32)
            hxd = jnp.dot(xdv, wg_ref[...], preferred_element_type=F32)
            a = (jnp.dot(hxs, as_ref[...], preferred_element_type=F32)
                 + jnp.dot(hxd, ad_ref[...], preferred_element_type=F32))
            exv = jnp.exp(_leaky(a, 0.2))
            m_ref[...] = hxs * exv
            ex_ref[...] = exv
        else:
            h1_ref, st_ref = outs
        h1_ref[...] = h1

        @pl.when(pl.program_id(0) == 0)
        def _():
            st_ref[...] = jnp.zeros_like(st_ref)

        st_ref[...] += jnp.concatenate(
            [jnp.sum(h1, 0, keepdims=True), jnp.sum(h1 * h1, 0, keepdims=True)], 0)

    in_specs = [pl.BlockSpec((BE, ni), lambda i: (i, 0)),
                pl.BlockSpec((BE, ni), lambda i: (i, 0)),
                pl.BlockSpec((BE, nei), lambda i: (i, 0)),
                pl.BlockSpec((1, din), lambda i: (0, 0)),
                pl.BlockSpec((1, din), lambda i: (0, 0)),
                pl.BlockSpec((din, NO), lambda i: (0, 0)),
                pl.BlockSpec((1, NO), lambda i: (0, 0)),
                pl.BlockSpec((ni, NO), lambda i: (0, 0)),
                pl.BlockSpec((NO, 1), lambda i: (0, 0)),
                pl.BlockSpec((NO, 1), lambda i: (0, 0))]
    if want_gat:
        out_specs = [pl.BlockSpec((BE, NO), lambda i: (i, 0)),
                     pl.BlockSpec((BE, NO), lambda i: (i, 0)),
                     pl.BlockSpec((BE, 1), lambda i: (i, 0)),
                     pl.BlockSpec((2, NO), lambda i: (0, 0))]
        out_shape = (jax.ShapeDtypeStruct((E, NO), F32),
                     jax.ShapeDtypeStruct((E, NO), F32),
                     jax.ShapeDtypeStruct((E, 1), F32),
                     jax.ShapeDtypeStruct((2, NO), F32))
    else:
        out_specs = [pl.BlockSpec((BE, NO), lambda i: (i, 0)),
                     pl.BlockSpec((2, NO), lambda i: (0, 0))]
        out_shape = (jax.ShapeDtypeStruct((E, NO), F32),
                     jax.ShapeDtypeStruct((2, NO), F32))
    return pl.pallas_call(
        body, grid=(T,), in_specs=in_specs, out_specs=out_specs,
        out_shape=out_shape)(xs, xd, e, sc3, sh3, W1, b1, Wg, asr, adr)


def _edge_mm(h, sc, sh, W, b, do_leaky, do_stats, head=None):
    T = E // BE
    Din = h.shape[1]
    Dout = W.shape[1] if head is None else head[0].shape[1]

    def body(h_ref, sc_ref, sh_ref, w_ref, b_ref, *rest):
        if head is None:
            o_ref = rest[0]
            st = rest[1:]
        else:
            wp_ref, bp_ref, o_ref = rest[0], rest[1], rest[2]
            st = rest[3:]
        hn = h_ref[...] * sc_ref[...] + sh_ref[...]
        z = jnp.dot(hn, w_ref[...], preferred_element_type=F32) + b_ref[...]
        if do_leaky:
            z = _leaky(z, LEAK)
        if head is not None:
            z = jnp.dot(z, wp_ref[...], preferred_element_type=F32) + bp_ref[...]
        o_ref[...] = z
        if do_stats:
            st_ref = st[0]

            @pl.when(pl.program_id(0) == 0)
            def _():
                st_ref[...] = jnp.zeros_like(st_ref)

            st_ref[...] += jnp.concatenate(
                [jnp.sum(z, 0, keepdims=True), jnp.sum(z * z, 0, keepdims=True)], 0)

    in_specs = [pl.BlockSpec((BE, Din), lambda i: (i, 0)),
                pl.BlockSpec((1, Din), lambda i: (0, 0)),
                pl.BlockSpec((1, Din), lambda i: (0, 0)),
                pl.BlockSpec((Din, W.shape[1]), lambda i: (0, 0)),
                pl.BlockSpec((1, W.shape[1]), lambda i: (0, 0))]
    args = [h, sc, sh, W, b]
    if head is not None:
        in_specs += [pl.BlockSpec(head[0].shape, lambda i: (0, 0)),
                     pl.BlockSpec((1, Dout), lambda i: (0, 0))]
        args += [head[0], head[1]]
    out_specs = [pl.BlockSpec((BE, Dout), lambda i: (i, 0))]
    out_shape = [jax.ShapeDtypeStruct((E, Dout), F32)]
    if do_stats:
        out_specs.append(pl.BlockSpec((2, Dout), lambda i: (0, 0)))
        out_shape.append(jax.ShapeDtypeStruct((2, Dout), F32))
    r = pl.pallas_call(
        body, grid=(T,),
        in_specs=in_specs,
        out_specs=out_specs,
        out_shape=tuple(out_shape))(*args)
    return r if do_stats else r[0]


def _bn_coefs(csum, csq, g, b, n):
    mu = csum / n
    var = csq / n - mu * mu
    s = g * lax.rsqrt(var + EPS)
    t = b - mu * s
    return s[None, :], t[None, :]


def kernel(node_features, edge_indices, edge_features, xbatch, params):
    p = params
    src = edge_indices[0].astype(jnp.int32)
    dst = edge_indices[1].astype(jnp.int32)
    src2 = src.reshape(NWIN, GW)
    dst2 = dst.reshape(NWIN, GW)
    zrows = jnp.zeros((NPAD // 16, NO), F32)

    cs_p, cd_p = _sc_counts(src, dst)
    cnt2 = _cnt_sum(cs_p, cd_p)[:, :N].T
    estats = _colstats(edge_features)

    e = edge_features
    n0stats = _colstats(node_features, rows=BN)
    xp, hx, es, xstats = _bn_project(
        node_features, n0stats, p['bn_node_g0'][None], p['bn_node_b0'][None],
        p['gat_W0'], p['gat_asrc0'][:, None], p['gat_adst0'][:, None], cnt2,
        want_gat=True)

    for i in range(3):
        ni = node_features.shape[1] if i == 0 else NO
        nei = edge_features.shape[1] if i == 0 else NO
        last = i == 2

        csum = jnp.concatenate([xstats[0], xstats[2], estats[0]])
        csq = jnp.concatenate([xstats[1], xstats[3], estats[1]])
        sc1, sh1 = _bn_coefs(csum, csq, p[f'e_bn1_g{i}'], p[f'e_bn1_b{i}'], E)

        xs, xd = _sc_gather(xp, src2, dst2)
        r = _edge_mlp1(xs, xd, e, sc1, sh1, p[f'e_lin1_W{i}'],
                       p[f'e_lin1_b{i}'][None], p[f'gat_W{i}'],
                       p[f'gat_asrc{i}'][:, None], p[f'gat_adst{i}'][:, None],
                       ni, nei, want_gat=not last)
        if last:
            h1, h1st = r
        else:
            h1, m, ex, h1st = r

        sc2, sh2 = _bn_coefs(h1st[0], h1st[1], p[f'e_bn2_g{i}'],
                             p[f'e_bn2_b{i}'], E)
        h2, h2st = _edge_mm(h1, sc2, sh2, p[f'e_lin2_W{i}'],
                            p[f'e_lin2_b{i}'][None], True, True)

        sc3, sh3 = _bn_coefs(h2st[0], h2st[1], p[f'e_bn3_g{i}'],
                             p[f'e_bn3_b{i}'], E)
        if last:
            return _edge_mm(h2, sc3, sh3, p[f'e_lin3_W{i}'],
                            p[f'e_lin3_b{i}'][None], False, False,
                            head=(p['pred_W'], p['pred_b'][None, :]))

        e, estats = _edge_mm(h2, sc3, sh3, p[f'e_lin3_W{i}'],
                             p[f'e_lin3_b{i}'][None], False, True)

        part = _sc_scatter_rows(m, dst2, zrows)
        denp = _sc_scatter_den(ex.reshape(E), dst)
        x, nstats = _gat_reduce(part, denp.T, hx, es,
                                p[f'gat_bias{i}'][None])

        r = _bn_project(x, nstats, p[f'bn_node_g{i + 1}'][None],
                        p[f'bn_node_b{i + 1}'][None], p[f'gat_W{i + 1}'],
                        p[f'gat_asrc{i + 1}'][:, None],
                        p[f'gat_adst{i + 1}'][:, None], cnt2,
                        want_gat=(i + 1 < 2))
        if i + 1 < 2:
            xp, hx, es, xstats = r
        else:
            xp, xstats = r
